# baseline (speedup 1.0000x reference)
import functools

import jax
import jax.numpy as jnp
from jax import lax
from jax.experimental import pallas as pl
from jax.experimental.pallas import tpu as pltpu
from jax.experimental.pallas import tpu_sc as plsc

N = 10000
D = 256
DH = 128
E = 160000
DE = 16
EPS = 1e-5

NS = 16
CH = 64
NCH = 160
NCH_HALF = 80
IDXB = 20
RING = 4
EPAD = NS * NCH * CH
RPT = 632
NPAD = NS * RPT

ROW_BLK = 1000
NB = N // ROW_BLK


def _sc_aggregate(xtA, src2A, dst4A, eaA, xtB, src2B, dst4B, eaB, zx, ze):
    mesh = plsc.VectorSubcoreMesh(core_axis_name="c", subcore_axis_name="s")
    f32 = jnp.float32

    @functools.partial(
        pl.kernel,
        out_type=[
            jax.ShapeDtypeStruct((2, NPAD, DH), f32),
            jax.ShapeDtypeStruct((2, NPAD, DE), f32),
            jax.ShapeDtypeStruct((2, NPAD, DH), f32),
            jax.ShapeDtypeStruct((2, NPAD, DE), f32),
        ],
        mesh=mesh,
        compiler_params=pltpu.CompilerParams(use_tc_tiling_on_sc=False),
        scratch_types=[
            pltpu.VMEM_SHARED((NPAD, DH), f32),
            pltpu.VMEM_SHARED((NPAD, DE), f32),
            pltpu.VMEM((IDXB, CH), jnp.int32),
            pltpu.VMEM((IDXB, CH), jnp.int32),
            [pltpu.VMEM((CH, DH), f32)] * RING,
            [pltpu.VMEM((CH, DE), f32)] * RING,
            [pltpu.SemaphoreType.DMA] * RING,
        ],
    )
    def body(xtA_h, src2A_h, dst4A_h, eaA_h, xtB_h, src2B_h, dst4B_h, eaB_h,
             zx_h, ze_h, outxA, outeA, outxB, outeB,
             acc_x, acc_e, idx_src, idx_dst, rows, eab, gsem):
        c = lax.axis_index("c")
        s = lax.axis_index("s")
        rbase = s * RPT

        def run_graph(xt_h, src2_h, dst4_h, ea_h, outx, oute):
            pltpu.sync_copy(zx_h, acc_x.at[pl.ds(rbase, RPT)])
            pltpu.sync_copy(ze_h, acc_e.at[pl.ds(rbase, RPT)])
            plsc.subcore_barrier()

            def pipe_block(load_fn, bufs, scat_ref):
                d_g = [None] * RING
                for k in range(RING - 1):
                    d_g[k] = load_fn(k, bufs[k], gsem[k])
                for jj in range(IDXB):
                    sl = jj % RING
                    d_g[sl].wait()
                    nxt = jj + RING - 1
                    if nxt < IDXB:
                        nsl = nxt % RING
                        d_g[nsl] = load_fn(nxt, bufs[nsl], gsem[nsl])
                    pltpu.sync_copy(bufs[sl], scat_ref.at[idx_dst.at[jj]],
                                    add=True)

            def blk(bi, carry):
                pltpu.sync_copy(src2_h.at[c, s, pl.ds(bi * IDXB, IDXB)],
                                idx_src)
                pltpu.sync_copy(dst4_h.at[s, pl.ds(bi * IDXB, IDXB)], idx_dst)
                pipe_block(
                    lambda jj, buf, sem: pltpu.async_copy(
                        xt_h.at[idx_src.at[jj]], buf, sem),
                    rows, acc_x)
                return carry

            lax.fori_loop(0, NCH // IDXB, blk, 0)

            def eblk(bi, carry):
                base = c * NCH_HALF + bi * IDXB
                pltpu.sync_copy(dst4_h.at[s, pl.ds(base, IDXB)], idx_dst)
                pipe_block(
                    lambda jj, buf, sem: pltpu.async_copy(
                        ea_h.at[s, base + jj], buf, sem),
                    eab, acc_e)
                return carry

            lax.fori_loop(0, NCH_HALF // IDXB, eblk, 0)
            plsc.subcore_barrier()
            pltpu.sync_copy(acc_x.at[pl.ds(rbase, RPT)],
                            outx.at[c, pl.ds(rbase, RPT)])
            pltpu.sync_copy(acc_e.at[pl.ds(rbase, RPT)],
                            oute.at[c, pl.ds(rbase, RPT)])
            plsc.subcore_barrier()

        run_graph(xtA_h, src2A_h, dst4A_h, eaA_h, outxA, outeA)
        run_graph(xtB_h, src2B_h, dst4B_h, eaB_h, outxB, outeB)

    return body(xtA, src2A, dst4A, eaA, xtB, src2B, dst4B, eaB, zx, ze)


def _prep_graph(x, edge_index, edge_attr):
    src = edge_index[0]
    dst = edge_index[1]
    pad = EPAD - E
    srcp = jnp.pad(src, (0, pad))
    dstp = jnp.pad(dst, (0, pad), constant_values=N)
    src2 = jnp.stack([2 * srcp, 2 * srcp + 1]).reshape(2, NS, NCH, CH)
    dst4 = dstp.reshape(NS, NCH, CH)
    ea = jnp.pad(edge_attr, ((0, pad), (0, 0))).reshape(NS, NCH, CH, DE)
    xt = x.reshape(2 * N, DH)
    return xt, src2, dst4, ea


def _assemble(x, sx, se, w_root, w_lo, w_hi, w_edge, b2):
    def asm_body(x_ref, sx_ref, se_ref, wr_ref, wlo_ref, whi_ref, we_ref,
                 b_ref, h_ref, stats_ref, acc_ref):
        i = pl.program_id(0)
        sesum = se_ref[0] + se_ref[1]
        h = (b_ref[...]
             + jnp.dot(x_ref[...], wr_ref[...],
                       preferred_element_type=jnp.float32)
             + jnp.dot(sx_ref[0], wlo_ref[...],
                       preferred_element_type=jnp.float32)
             + jnp.dot(sx_ref[1], whi_ref[...],
                       preferred_element_type=jnp.float32)
             + jnp.dot(sesum, we_ref[...],
                       preferred_element_type=jnp.float32))
        h_ref[...] = h

        @pl.when(i == 0)
        def _():
            acc_ref[...] = jnp.zeros_like(acc_ref)

        acc_ref[0:1] += jnp.sum(h, axis=0, keepdims=True)
        acc_ref[1:2] += jnp.sum(h * h, axis=0, keepdims=True)

        @pl.when(i == NB - 1)
        def _():
            stats_ref[...] = acc_ref[...]

    return pl.pallas_call(
        asm_body,
        grid=(NB,),
        in_specs=[
            pl.BlockSpec((ROW_BLK, D), lambda i: (i, 0)),
            pl.BlockSpec((2, ROW_BLK, DH), lambda i: (0, i, 0)),
            pl.BlockSpec((2, ROW_BLK, DE), lambda i: (0, i, 0)),
            pl.BlockSpec((D, D), lambda i: (0, 0)),
            pl.BlockSpec((DH, D), lambda i: (0, 0)),
            pl.BlockSpec((DH, D), lambda i: (0, 0)),
            pl.BlockSpec((DE, D), lambda i: (0, 0)),
            pl.BlockSpec((1, D), lambda i: (0, 0)),
        ],
        out_specs=[
            pl.BlockSpec((ROW_BLK, D), lambda i: (i, 0)),
            pl.BlockSpec((2, D), lambda i: (0, 0)),
        ],
        out_shape=[
            jax.ShapeDtypeStruct((N, D), jnp.float32),
            jax.ShapeDtypeStruct((2, D), jnp.float32),
        ],
        scratch_shapes=[pltpu.VMEM((2, D), jnp.float32)],
    )(x, sx, se, w_root, w_lo, w_hi, w_edge, b2)


def _normalize(h, stats, gamma2, beta2):
    def norm_body(h_ref, st_ref, g_ref, be_ref, o_ref):
        mean = st_ref[0:1] * (1.0 / N)
        var = st_ref[1:2] * (1.0 / N) - mean * mean
        inv = lax.rsqrt(var + EPS)
        o_ref[...] = jnp.maximum(
            (h_ref[...] - mean) * (inv * g_ref[...]) + be_ref[...], 0.0)

    return pl.pallas_call(
        norm_body,
        grid=(NB,),
        in_specs=[
            pl.BlockSpec((ROW_BLK, D), lambda i: (i, 0)),
            pl.BlockSpec((2, D), lambda i: (0, 0)),
            pl.BlockSpec((1, D), lambda i: (0, 0)),
            pl.BlockSpec((1, D), lambda i: (0, 0)),
        ],
        out_specs=pl.BlockSpec((ROW_BLK, D), lambda i: (i, 0)),
        out_shape=jax.ShapeDtypeStruct((N, D), jnp.float32),
    )(h, stats, gamma2, beta2)


def kernel(xA, edge_indexA, edge_attrA, xB, edge_indexB, edge_attrB,
           W_root, W_src, W_edge, b, gamma, beta):
    xtA, src2A, dst4A, eaA = _prep_graph(xA, edge_indexA, edge_attrA)
    xtB, src2B, dst4B, eaB = _prep_graph(xB, edge_indexB, edge_attrB)
    zx = jnp.zeros((RPT, DH), jnp.float32)
    ze = jnp.zeros((RPT, DE), jnp.float32)

    sxA, seA, sxB, seB = _sc_aggregate(
        xtA, src2A, dst4A, eaA, xtB, src2B, dst4B, eaB, zx, ze)

    w_lo = W_src[:DH]
    w_hi = W_src[DH:]
    b2 = b.reshape(1, D)
    gamma2 = gamma.reshape(1, D)
    beta2 = beta.reshape(1, D)

    hA, statsA = _assemble(xA, sxA, seA, W_root, w_lo, w_hi, W_edge, b2)
    hB, statsB = _assemble(xB, sxB, seB, W_root, w_lo, w_hi, W_edge, b2)
    outA = _normalize(hA, statsA, gamma2, beta2)
    outB = _normalize(hB, statsB, gamma2, beta2)
    return (outA, outB)


# speedup vs baseline: 1.0327x; 1.0327x over previous
import functools

import jax
import jax.numpy as jnp
from jax import lax
from jax.experimental import pallas as pl
from jax.experimental.pallas import tpu as pltpu
from jax.experimental.pallas import tpu_sc as plsc

N = 10000
D = 256
DH = 128
E = 160000
DE = 16
EPS = 1e-5

NS = 16
CH = 64
NCH = 160
NCH_HALF = 80
IDXB = 20
RING = 4
EPAD = NS * NCH * CH
RPT = 632
NPAD = NS * RPT

ROW_BLK = 1000
NB = N // ROW_BLK


def _sc_aggregate(xt, src2, dst4, ea, zx, ze):
    mesh = plsc.VectorSubcoreMesh(core_axis_name="c", subcore_axis_name="s")
    f32 = jnp.float32

    @functools.partial(
        pl.kernel,
        out_type=[
            jax.ShapeDtypeStruct((2, NPAD, DH), f32),
            jax.ShapeDtypeStruct((2, NPAD, DE), f32),
        ],
        mesh=mesh,
        compiler_params=pltpu.CompilerParams(use_tc_tiling_on_sc=False),
        scratch_types=[
            pltpu.VMEM_SHARED((NPAD, DH), f32),
            pltpu.VMEM_SHARED((NPAD, DE), f32),
            pltpu.VMEM((IDXB, CH), jnp.int32),
            pltpu.VMEM((IDXB, CH), jnp.int32),
            [pltpu.VMEM((CH, DH), f32)] * RING,
            [pltpu.VMEM((CH, DE), f32)] * RING,
            [pltpu.SemaphoreType.DMA] * RING,
        ],
    )
    def body(xt_g, src2_g, dst4_g, ea_g, zx_h, ze_h, outx_g, oute_g,
             acc_x, acc_e, idx_src, idx_dst, rows, eab, gsem):
        c = lax.axis_index("c")
        s = lax.axis_index("s")
        rbase = s * RPT

        def run_graph(xt_h, src2_h, dst4_h, ea_h, outx, oute):
            pltpu.sync_copy(zx_h, acc_x.at[pl.ds(rbase, RPT)])
            pltpu.sync_copy(ze_h, acc_e.at[pl.ds(rbase, RPT)])
            plsc.subcore_barrier()

            def pipe_block(load_fn, bufs, scat_ref):
                d_g = [None] * RING
                for k in range(RING - 1):
                    d_g[k] = load_fn(k, bufs[k], gsem[k])
                for jj in range(IDXB):
                    sl = jj % RING
                    d_g[sl].wait()
                    nxt = jj + RING - 1
                    if nxt < IDXB:
                        nsl = nxt % RING
                        d_g[nsl] = load_fn(nxt, bufs[nsl], gsem[nsl])
                    pltpu.sync_copy(bufs[sl], scat_ref.at[idx_dst.at[jj]],
                                    add=True)

            def blk(bi, carry):
                pltpu.sync_copy(src2_h.at[c, s, pl.ds(bi * IDXB, IDXB)],
                                idx_src)
                pltpu.sync_copy(dst4_h.at[s, pl.ds(bi * IDXB, IDXB)], idx_dst)
                pipe_block(
                    lambda jj, buf, sem: pltpu.async_copy(
                        xt_h.at[idx_src.at[jj]], buf, sem),
                    rows, acc_x)
                return carry

            lax.fori_loop(0, NCH // IDXB, blk, 0)

            def eblk(bi, carry):
                base = c * NCH_HALF + bi * IDXB
                pltpu.sync_copy(dst4_h.at[s, pl.ds(base, IDXB)], idx_dst)
                pipe_block(
                    lambda jj, buf, sem: pltpu.async_copy(
                        ea_h.at[s, base + jj], buf, sem),
                    eab, acc_e)
                return carry

            lax.fori_loop(0, NCH_HALF // IDXB, eblk, 0)
            plsc.subcore_barrier()
            pltpu.sync_copy(acc_x.at[pl.ds(rbase, RPT)],
                            outx.at[c, pl.ds(rbase, RPT)])
            pltpu.sync_copy(acc_e.at[pl.ds(rbase, RPT)],
                            oute.at[c, pl.ds(rbase, RPT)])
            plsc.subcore_barrier()

        run_graph(xt_g, src2_g, dst4_g, ea_g, outx_g, oute_g)

    return body(xt, src2, dst4, ea, zx, ze)


def _prep_graph(x, edge_index, edge_attr):
    src = edge_index[0]
    dst = edge_index[1]
    pad = EPAD - E
    srcp = jnp.pad(src, (0, pad))
    dstp = jnp.pad(dst, (0, pad), constant_values=N)
    src2 = jnp.stack([2 * srcp, 2 * srcp + 1]).reshape(2, NS, NCH, CH)
    dst4 = dstp.reshape(NS, NCH, CH)
    ea = jnp.pad(edge_attr, ((0, pad), (0, 0))).reshape(NS, NCH, CH, DE)
    xt = x.reshape(2 * N, DH)
    return xt, src2, dst4, ea


def _assemble(x, sx, se, w_root, w_lo, w_hi, w_edge, b2):
    def asm_body(x_ref, sx_ref, se_ref, wr_ref, wlo_ref, whi_ref, we_ref,
                 b_ref, h_ref, stats_ref, acc_ref):
        i = pl.program_id(0)
        sesum = se_ref[0] + se_ref[1]
        h = (b_ref[...]
             + jnp.dot(x_ref[...], wr_ref[...],
                       preferred_element_type=jnp.float32)
             + jnp.dot(sx_ref[0], wlo_ref[...],
                       preferred_element_type=jnp.float32)
             + jnp.dot(sx_ref[1], whi_ref[...],
                       preferred_element_type=jnp.float32)
             + jnp.dot(sesum, we_ref[...],
                       preferred_element_type=jnp.float32))
        h_ref[...] = h

        @pl.when(i == 0)
        def _():
            acc_ref[...] = jnp.zeros_like(acc_ref)

        acc_ref[0:1] += jnp.sum(h, axis=0, keepdims=True)
        acc_ref[1:2] += jnp.sum(h * h, axis=0, keepdims=True)

        @pl.when(i == NB - 1)
        def _():
            stats_ref[...] = acc_ref[...]

    return pl.pallas_call(
        asm_body,
        grid=(NB,),
        in_specs=[
            pl.BlockSpec((ROW_BLK, D), lambda i: (i, 0)),
            pl.BlockSpec((2, ROW_BLK, DH), lambda i: (0, i, 0)),
            pl.BlockSpec((2, ROW_BLK, DE), lambda i: (0, i, 0)),
            pl.BlockSpec((D, D), lambda i: (0, 0)),
            pl.BlockSpec((DH, D), lambda i: (0, 0)),
            pl.BlockSpec((DH, D), lambda i: (0, 0)),
            pl.BlockSpec((DE, D), lambda i: (0, 0)),
            pl.BlockSpec((1, D), lambda i: (0, 0)),
        ],
        out_specs=[
            pl.BlockSpec((ROW_BLK, D), lambda i: (i, 0)),
            pl.BlockSpec((2, D), lambda i: (0, 0)),
        ],
        out_shape=[
            jax.ShapeDtypeStruct((N, D), jnp.float32),
            jax.ShapeDtypeStruct((2, D), jnp.float32),
        ],
        scratch_shapes=[pltpu.VMEM((2, D), jnp.float32)],
    )(x, sx, se, w_root, w_lo, w_hi, w_edge, b2)


def _normalize(h, stats, gamma2, beta2):
    def norm_body(h_ref, st_ref, g_ref, be_ref, o_ref):
        mean = st_ref[0:1] * (1.0 / N)
        var = st_ref[1:2] * (1.0 / N) - mean * mean
        inv = lax.rsqrt(var + EPS)
        o_ref[...] = jnp.maximum(
            (h_ref[...] - mean) * (inv * g_ref[...]) + be_ref[...], 0.0)

    return pl.pallas_call(
        norm_body,
        grid=(NB,),
        in_specs=[
            pl.BlockSpec((ROW_BLK, D), lambda i: (i, 0)),
            pl.BlockSpec((2, D), lambda i: (0, 0)),
            pl.BlockSpec((1, D), lambda i: (0, 0)),
            pl.BlockSpec((1, D), lambda i: (0, 0)),
        ],
        out_specs=pl.BlockSpec((ROW_BLK, D), lambda i: (i, 0)),
        out_shape=jax.ShapeDtypeStruct((N, D), jnp.float32),
    )(h, stats, gamma2, beta2)


def kernel(xA, edge_indexA, edge_attrA, xB, edge_indexB, edge_attrB,
           W_root, W_src, W_edge, b, gamma, beta):
    xtA, src2A, dst4A, eaA = _prep_graph(xA, edge_indexA, edge_attrA)
    xtB, src2B, dst4B, eaB = _prep_graph(xB, edge_indexB, edge_attrB)
    zx = jnp.zeros((RPT, DH), jnp.float32)
    ze = jnp.zeros((RPT, DE), jnp.float32)

    sxA, seA = _sc_aggregate(xtA, src2A, dst4A, eaA, zx, ze)
    sxB, seB = _sc_aggregate(xtB, src2B, dst4B, eaB, zx, ze)

    w_lo = W_src[:DH]
    w_hi = W_src[DH:]
    b2 = b.reshape(1, D)
    gamma2 = gamma.reshape(1, D)
    beta2 = beta.reshape(1, D)

    hA, statsA = _assemble(xA, sxA, seA, W_root, w_lo, w_hi, W_edge, b2)
    hB, statsB = _assemble(xB, sxB, seB, W_root, w_lo, w_hi, W_edge, b2)
    outA = _normalize(hA, statsA, gamma2, beta2)
    outB = _normalize(hB, statsB, gamma2, beta2)
    return (outA, outB)


# speedup vs baseline: 1.0358x; 1.0030x over previous
import functools

import jax
import jax.numpy as jnp
from jax import lax
from jax.experimental import pallas as pl
from jax.experimental.pallas import tpu as pltpu
from jax.experimental.pallas import tpu_sc as plsc

N = 10000
D = 256
DH = 128
E = 160000
DE = 16
EPS = 1e-5

NS = 16
CH = 40
NCH = 256
NCH_HALF = 128
IDXB = 32
RING = 6
EPAD = NS * NCH * CH
RPT = 632
NPAD = NS * RPT

ROW_BLK = 1000
NB = N // ROW_BLK


def _sc_aggregate(xt, src2, dst4, ea, zx, ze):
    mesh = plsc.VectorSubcoreMesh(core_axis_name="c", subcore_axis_name="s")
    f32 = jnp.float32

    @functools.partial(
        pl.kernel,
        out_type=[
            jax.ShapeDtypeStruct((2, NPAD, DH), f32),
            jax.ShapeDtypeStruct((2, NPAD, DE), f32),
        ],
        mesh=mesh,
        compiler_params=pltpu.CompilerParams(use_tc_tiling_on_sc=False),
        scratch_types=[
            pltpu.VMEM_SHARED((NPAD, DH), f32),
            pltpu.VMEM_SHARED((NPAD, DE), f32),
            pltpu.VMEM((IDXB, CH), jnp.int32),
            pltpu.VMEM((IDXB, CH), jnp.int32),
            [pltpu.VMEM((CH, DH), f32)] * RING,
            [pltpu.VMEM((CH, DE), f32)] * RING,
            [pltpu.SemaphoreType.DMA] * RING,
        ],
    )
    def body(xt_g, src2_g, dst4_g, ea_g, zx_h, ze_h, outx_g, oute_g,
             acc_x, acc_e, idx_src, idx_dst, rows, eab, gsem):
        c = lax.axis_index("c")
        s = lax.axis_index("s")
        rbase = s * RPT

        def run_graph(xt_h, src2_h, dst4_h, ea_h, outx, oute):
            pltpu.sync_copy(zx_h, acc_x.at[pl.ds(rbase, RPT)])
            pltpu.sync_copy(ze_h, acc_e.at[pl.ds(rbase, RPT)])
            plsc.subcore_barrier()

            def pipe_block(load_fn, bufs, scat_ref):
                d_g = [None] * RING
                for k in range(RING - 1):
                    d_g[k] = load_fn(k, bufs[k], gsem[k])
                for jj in range(IDXB):
                    sl = jj % RING
                    d_g[sl].wait()
                    nxt = jj + RING - 1
                    if nxt < IDXB:
                        nsl = nxt % RING
                        d_g[nsl] = load_fn(nxt, bufs[nsl], gsem[nsl])
                    pltpu.sync_copy(bufs[sl], scat_ref.at[idx_dst.at[jj]],
                                    add=True)

            def blk(bi, carry):
                pltpu.sync_copy(src2_h.at[c, s, pl.ds(bi * IDXB, IDXB)],
                                idx_src)
                pltpu.sync_copy(dst4_h.at[s, pl.ds(bi * IDXB, IDXB)], idx_dst)
                pipe_block(
                    lambda jj, buf, sem: pltpu.async_copy(
                        xt_h.at[idx_src.at[jj]], buf, sem),
                    rows, acc_x)
                return carry

            lax.fori_loop(0, NCH // IDXB, blk, 0)

            def eblk(bi, carry):
                base = c * NCH_HALF + bi * IDXB
                pltpu.sync_copy(dst4_h.at[s, pl.ds(base, IDXB)], idx_dst)
                pipe_block(
                    lambda jj, buf, sem: pltpu.async_copy(
                        ea_h.at[s, base + jj], buf, sem),
                    eab, acc_e)
                return carry

            lax.fori_loop(0, NCH_HALF // IDXB, eblk, 0)
            plsc.subcore_barrier()
            pltpu.sync_copy(acc_x.at[pl.ds(rbase, RPT)],
                            outx.at[c, pl.ds(rbase, RPT)])
            pltpu.sync_copy(acc_e.at[pl.ds(rbase, RPT)],
                            oute.at[c, pl.ds(rbase, RPT)])
            plsc.subcore_barrier()

        run_graph(xt_g, src2_g, dst4_g, ea_g, outx_g, oute_g)

    return body(xt, src2, dst4, ea, zx, ze)


def _prep_graph(x, edge_index, edge_attr):
    src = edge_index[0]
    dst = edge_index[1]
    pad = EPAD - E
    srcp = jnp.pad(src, (0, pad))
    dstp = jnp.pad(dst, (0, pad), constant_values=N)
    src2 = jnp.stack([2 * srcp, 2 * srcp + 1]).reshape(2, NS, NCH, CH)
    dst4 = dstp.reshape(NS, NCH, CH)
    ea = jnp.pad(edge_attr, ((0, pad), (0, 0))).reshape(NS, NCH, CH, DE)
    xt = x.reshape(2 * N, DH)
    return xt, src2, dst4, ea


def _assemble(x, sx, se, w_root, w_lo, w_hi, w_edge, b2):
    def asm_body(x_ref, sx_ref, se_ref, wr_ref, wlo_ref, whi_ref, we_ref,
                 b_ref, h_ref, stats_ref, acc_ref):
        i = pl.program_id(0)
        sesum = se_ref[0] + se_ref[1]
        h = (b_ref[...]
             + jnp.dot(x_ref[...], wr_ref[...],
                       preferred_element_type=jnp.float32)
             + jnp.dot(sx_ref[0], wlo_ref[...],
                       preferred_element_type=jnp.float32)
             + jnp.dot(sx_ref[1], whi_ref[...],
                       preferred_element_type=jnp.float32)
             + jnp.dot(sesum, we_ref[...],
                       preferred_element_type=jnp.float32))
        h_ref[...] = h

        @pl.when(i == 0)
        def _():
            acc_ref[...] = jnp.zeros_like(acc_ref)

        acc_ref[0:1] += jnp.sum(h, axis=0, keepdims=True)
        acc_ref[1:2] += jnp.sum(h * h, axis=0, keepdims=True)

        @pl.when(i == NB - 1)
        def _():
            stats_ref[...] = acc_ref[...]

    return pl.pallas_call(
        asm_body,
        grid=(NB,),
        in_specs=[
            pl.BlockSpec((ROW_BLK, D), lambda i: (i, 0)),
            pl.BlockSpec((2, ROW_BLK, DH), lambda i: (0, i, 0)),
            pl.BlockSpec((2, ROW_BLK, DE), lambda i: (0, i, 0)),
            pl.BlockSpec((D, D), lambda i: (0, 0)),
            pl.BlockSpec((DH, D), lambda i: (0, 0)),
            pl.BlockSpec((DH, D), lambda i: (0, 0)),
            pl.BlockSpec((DE, D), lambda i: (0, 0)),
            pl.BlockSpec((1, D), lambda i: (0, 0)),
        ],
        out_specs=[
            pl.BlockSpec((ROW_BLK, D), lambda i: (i, 0)),
            pl.BlockSpec((2, D), lambda i: (0, 0)),
        ],
        out_shape=[
            jax.ShapeDtypeStruct((N, D), jnp.float32),
            jax.ShapeDtypeStruct((2, D), jnp.float32),
        ],
        scratch_shapes=[pltpu.VMEM((2, D), jnp.float32)],
    )(x, sx, se, w_root, w_lo, w_hi, w_edge, b2)


def _normalize(h, stats, gamma2, beta2):
    def norm_body(h_ref, st_ref, g_ref, be_ref, o_ref):
        mean = st_ref[0:1] * (1.0 / N)
        var = st_ref[1:2] * (1.0 / N) - mean * mean
        inv = lax.rsqrt(var + EPS)
        o_ref[...] = jnp.maximum(
            (h_ref[...] - mean) * (inv * g_ref[...]) + be_ref[...], 0.0)

    return pl.pallas_call(
        norm_body,
        grid=(NB,),
        in_specs=[
            pl.BlockSpec((ROW_BLK, D), lambda i: (i, 0)),
            pl.BlockSpec((2, D), lambda i: (0, 0)),
            pl.BlockSpec((1, D), lambda i: (0, 0)),
            pl.BlockSpec((1, D), lambda i: (0, 0)),
        ],
        out_specs=pl.BlockSpec((ROW_BLK, D), lambda i: (i, 0)),
        out_shape=jax.ShapeDtypeStruct((N, D), jnp.float32),
    )(h, stats, gamma2, beta2)


def kernel(xA, edge_indexA, edge_attrA, xB, edge_indexB, edge_attrB,
           W_root, W_src, W_edge, b, gamma, beta):
    xtA, src2A, dst4A, eaA = _prep_graph(xA, edge_indexA, edge_attrA)
    xtB, src2B, dst4B, eaB = _prep_graph(xB, edge_indexB, edge_attrB)
    zx = jnp.zeros((RPT, DH), jnp.float32)
    ze = jnp.zeros((RPT, DE), jnp.float32)

    sxA, seA = _sc_aggregate(xtA, src2A, dst4A, eaA, zx, ze)
    sxB, seB = _sc_aggregate(xtB, src2B, dst4B, eaB, zx, ze)

    w_lo = W_src[:DH]
    w_hi = W_src[DH:]
    b2 = b.reshape(1, D)
    gamma2 = gamma.reshape(1, D)
    beta2 = beta.reshape(1, D)

    hA, statsA = _assemble(xA, sxA, seA, W_root, w_lo, w_hi, W_edge, b2)
    hB, statsB = _assemble(xB, sxB, seB, W_root, w_lo, w_hi, W_edge, b2)
    outA = _normalize(hA, statsA, gamma2, beta2)
    outB = _normalize(hB, statsB, gamma2, beta2)
    return (outA, outB)


# speedup vs baseline: 1.0400x; 1.0041x over previous
import functools

import jax
import jax.numpy as jnp
from jax import lax
from jax.experimental import pallas as pl
from jax.experimental.pallas import tpu as pltpu
from jax.experimental.pallas import tpu_sc as plsc

N = 10000
D = 256
DH = 128
E = 160000
DE = 16
EPS = 1e-5

NS = 16
CH = 40
NCH = 256
NCH_HALF = 128
IDXB = 32
RING = 6
EPAD = NS * NCH * CH
RPT = 632
NPAD = NS * RPT

ROW_BLK = 1000
NB = N // ROW_BLK


def _sc_aggregate(xt, src2, dst4, ea, zx, ze):
    mesh = plsc.VectorSubcoreMesh(core_axis_name="c", subcore_axis_name="s")
    f32 = jnp.float32

    @functools.partial(
        pl.kernel,
        out_type=[
            jax.ShapeDtypeStruct((2, NPAD, DH), f32),
            jax.ShapeDtypeStruct((2, NPAD, DE), f32),
        ],
        mesh=mesh,
        compiler_params=pltpu.CompilerParams(use_tc_tiling_on_sc=False),
        scratch_types=[
            pltpu.VMEM_SHARED((NPAD, DH), f32),
            pltpu.VMEM_SHARED((NPAD, DE), f32),
            pltpu.VMEM((IDXB, CH), jnp.int32),
            pltpu.VMEM((IDXB, CH), jnp.int32),
            [pltpu.VMEM((CH, DH), f32)] * RING,
            [pltpu.VMEM((CH, DE), f32)] * RING,
            [pltpu.SemaphoreType.DMA] * RING,
        ],
    )
    def body(xt_g, src2_g, dst4_g, ea_g, zx_h, ze_h, outx_g, oute_g,
             acc_x, acc_e, idx_src, idx_dst, rows, eab, gsem):
        c = lax.axis_index("c")
        s = lax.axis_index("s")
        rbase = s * RPT

        def run_graph(xt_h, src2_h, dst4_h, ea_h, outx, oute):
            pltpu.sync_copy(zx_h, acc_x.at[pl.ds(rbase, RPT)])
            pltpu.sync_copy(ze_h, acc_e.at[pl.ds(rbase, RPT)])
            plsc.subcore_barrier()

            def pipe_block(load_fn, bufs, scat_ref):
                d_g = [None] * RING
                for k in range(RING - 1):
                    d_g[k] = load_fn(k, bufs[k], gsem[k])
                for jj in range(IDXB):
                    sl = jj % RING
                    d_g[sl].wait()
                    nxt = jj + RING - 1
                    if nxt < IDXB:
                        nsl = nxt % RING
                        d_g[nsl] = load_fn(nxt, bufs[nsl], gsem[nsl])
                    pltpu.sync_copy(bufs[sl], scat_ref.at[idx_dst.at[jj]],
                                    add=True)

            def blk(bi, carry):
                pltpu.sync_copy(src2_h.at[c, s, pl.ds(bi * IDXB, IDXB)],
                                idx_src)
                pltpu.sync_copy(dst4_h.at[s, pl.ds(bi * IDXB, IDXB)], idx_dst)
                pipe_block(
                    lambda jj, buf, sem: pltpu.async_copy(
                        xt_h.at[idx_src.at[jj]], buf, sem),
                    rows, acc_x)
                return carry

            lax.fori_loop(0, NCH // IDXB, blk, 0)

            def eblk(bi, carry):
                base = c * NCH_HALF + bi * IDXB
                pltpu.sync_copy(dst4_h.at[s, pl.ds(base, IDXB)], idx_dst)
                pipe_block(
                    lambda jj, buf, sem: pltpu.async_copy(
                        ea_h.at[s, base + jj], buf, sem),
                    eab, acc_e)
                return carry

            lax.fori_loop(0, NCH_HALF // IDXB, eblk, 0)
            plsc.subcore_barrier()
            pltpu.sync_copy(acc_x.at[pl.ds(rbase, RPT)],
                            outx.at[c, pl.ds(rbase, RPT)])
            pltpu.sync_copy(acc_e.at[pl.ds(rbase, RPT)],
                            oute.at[c, pl.ds(rbase, RPT)])
            plsc.subcore_barrier()

        run_graph(xt_g, src2_g, dst4_g, ea_g, outx_g, oute_g)

    return body(xt, src2, dst4, ea, zx, ze)


def _prep_graph(x, edge_index, edge_attr):
    src = edge_index[0]
    dst = edge_index[1]
    pad = EPAD - E
    srcp = jnp.pad(src, (0, pad))
    dstp = jnp.pad(dst, (0, pad), constant_values=N)
    src2 = jnp.stack([2 * srcp, 2 * srcp + 1]).reshape(2, NS, NCH, CH)
    dst4 = dstp.reshape(NS, NCH, CH)
    ea = jnp.pad(edge_attr, ((0, pad), (0, 0))).reshape(NS, NCH, CH, DE)
    xt = x.reshape(2 * N, DH)
    return xt, src2, dst4, ea


def _fused_tc(x, sx, se, w_root, w_lo, w_hi, w_edge, b2, gamma2, beta2):
    def body(x_ref, sx_ref, se_ref, wr_ref, wlo_ref, whi_ref, we_ref,
             b_ref, g_ref, be_ref, o_ref):
        sesum = se_ref[0] + se_ref[1]
        h = (b_ref[...]
             + jnp.dot(x_ref[...], wr_ref[...],
                       preferred_element_type=jnp.float32)
             + jnp.dot(sx_ref[0], wlo_ref[...],
                       preferred_element_type=jnp.float32)
             + jnp.dot(sx_ref[1], whi_ref[...],
                       preferred_element_type=jnp.float32)
             + jnp.dot(sesum, we_ref[...],
                       preferred_element_type=jnp.float32))
        mean = jnp.mean(h, axis=0, keepdims=True)
        var = jnp.mean(h * h, axis=0, keepdims=True) - mean * mean
        inv = lax.rsqrt(var + EPS)
        o_ref[...] = jnp.maximum((h - mean) * (inv * g_ref[...]) + be_ref[...],
                                 0.0)

    return pl.pallas_call(
        body,
        grid=(1,),
        in_specs=[
            pl.BlockSpec((N, D), lambda i: (0, 0)),
            pl.BlockSpec((2, N, DH), lambda i: (0, 0, 0)),
            pl.BlockSpec((2, N, DE), lambda i: (0, 0, 0)),
            pl.BlockSpec((D, D), lambda i: (0, 0)),
            pl.BlockSpec((DH, D), lambda i: (0, 0)),
            pl.BlockSpec((DH, D), lambda i: (0, 0)),
            pl.BlockSpec((DE, D), lambda i: (0, 0)),
            pl.BlockSpec((1, D), lambda i: (0, 0)),
            pl.BlockSpec((1, D), lambda i: (0, 0)),
            pl.BlockSpec((1, D), lambda i: (0, 0)),
        ],
        out_specs=pl.BlockSpec((N, D), lambda i: (0, 0)),
        out_shape=jax.ShapeDtypeStruct((N, D), jnp.float32),
    )(x, sx, se, w_root, w_lo, w_hi, w_edge, b2, gamma2, beta2)


def kernel(xA, edge_indexA, edge_attrA, xB, edge_indexB, edge_attrB,
           W_root, W_src, W_edge, b, gamma, beta):
    xtA, src2A, dst4A, eaA = _prep_graph(xA, edge_indexA, edge_attrA)
    xtB, src2B, dst4B, eaB = _prep_graph(xB, edge_indexB, edge_attrB)
    zx = jnp.zeros((RPT, DH), jnp.float32)
    ze = jnp.zeros((RPT, DE), jnp.float32)

    sxA, seA = _sc_aggregate(xtA, src2A, dst4A, eaA, zx, ze)
    sxB, seB = _sc_aggregate(xtB, src2B, dst4B, eaB, zx, ze)

    w_lo = W_src[:DH]
    w_hi = W_src[DH:]
    b2 = b.reshape(1, D)
    gamma2 = gamma.reshape(1, D)
    beta2 = beta.reshape(1, D)

    outA = _fused_tc(xA, sxA, seA, W_root, w_lo, w_hi, W_edge, b2,
                     gamma2, beta2)
    outB = _fused_tc(xB, sxB, seB, W_root, w_lo, w_hi, W_edge, b2,
                     gamma2, beta2)
    return (outA, outB)
